# speedup vs baseline: 1.6060x; 1.6060x over previous
import functools

import jax
import jax.numpy as jnp
from jax import lax
from jax.experimental import pallas as pl
from jax.experimental.pallas import tpu as pltpu
from jax.experimental.pallas import tpu_sc as plsc

_B, _N, _CODE_DIM, _K, _HIDDEN = 16, 1024, 256, 8192, 768
_COMMITMENT_COST = 0.25
_THRESHOLD = 0.5

_TOK = _B * _N


_info = plsc.get_sparse_core_info()
_NC, _NS = _info.num_cores, _info.num_subcores
_NW = _NC * _NS
_PER_W = _TOK // _NW
_CH = 128
_N_CH = _PER_W // _CH


def _make_sc_gather():
    mesh = plsc.VectorSubcoreMesh(core_axis_name="c", subcore_axis_name="s")

    @functools.partial(
        pl.kernel,
        mesh=mesh,
        out_type=jax.ShapeDtypeStruct((_TOK, _CODE_DIM), jnp.float32),
        scratch_types=[
            pltpu.VMEM((_N_CH, _CH), jnp.int32),
            pltpu.VMEM((_CH, _CODE_DIM), jnp.float32),
            pltpu.VMEM((_CH, _CODE_DIM), jnp.float32),
            pltpu.SemaphoreType.DMA,
            pltpu.SemaphoreType.DMA,
        ],
    )
    def sc_gather(table_hbm, idx_hbm, out_hbm, idx_v, rows0, rows1, sem0, sem1):
        wid = lax.axis_index("s") * _NC + lax.axis_index("c")
        base = wid * _PER_W
        pltpu.sync_copy(idx_hbm.at[wid], idx_v)
        bufs = (rows0, rows1)
        sems = (sem0, sem1)
        copies = [None, None]
        copies[0] = pltpu.async_copy(table_hbm.at[idx_v.at[0]], rows0, sem0)
        for c in range(_N_CH):
            cur = c % 2
            if c + 1 < _N_CH:
                nxt = (c + 1) % 2
                copies[nxt] = pltpu.async_copy(
                    table_hbm.at[idx_v.at[c + 1]], bufs[nxt], sems[nxt])
            copies[cur].wait()
            pltpu.sync_copy(bufs[cur], out_hbm.at[pl.ds(base + c * _CH, _CH)])

    return sc_gather


_sc_gather = _make_sc_gather()


_SUB = _N // 128


def _main_body(q_ref, m_ref, w_ref, b_ref, out_ref, valid_ref, loss_ref, acc_ref):
    bidx = pl.program_id(0)
    q = q_ref[0]
    m = m_ref[0]
    w = w_ref[...]
    bias = b_ref[...]

    out = lax.dot_general(q, w, (((1,), (1,)), ((), ())),
                          preferred_element_type=jnp.float32) + bias
    out_ref[0] = out

    q3 = q.reshape(_SUB, 128, _CODE_DIM)
    m3 = m.reshape(_SUB, 128, _CODE_DIM)
    sim = jnp.sum(q3 * m3, axis=2)
    sq = jnp.sum((m3 - q3) ** 2, axis=2)

    neg = jnp.float32(-jnp.inf)
    cur = jnp.float32(jnp.inf)
    removed = jnp.float32(0.0)
    for _ in range(5):
        mmax = jnp.max(jnp.where(sim < cur, sim, neg))
        cnt_eq = jnp.sum(jnp.where(sim == mmax, 1.0, 0.0))
        upd = removed < 5.0
        removed = jnp.where(upd, removed + cnt_eq, removed)
        cur = jnp.where(upd, mmax, cur)

    thresh = jnp.minimum(cur, jnp.float32(_THRESHOLD))
    validf = (sim >= thresh).astype(jnp.float32)
    valid_ref[0] = validf.astype(jnp.int32)

    num = jnp.sum(sq * validf)
    cnt = jnp.sum(validf)

    @pl.when(bidx == 0)
    def _init():
        acc_ref[0] = num
        acc_ref[1] = cnt

    @pl.when(bidx > 0)
    def _accum():
        acc_ref[0] = acc_ref[0] + num
        acc_ref[1] = acc_ref[1] + cnt

    @pl.when(bidx == _B - 1)
    def _final():
        denom = acc_ref[1] * jnp.float32(_CODE_DIM)
        loss = (1.0 + _COMMITMENT_COST) * acc_ref[0] / denom
        loss_ref[...] = jnp.full((1, 1), loss, jnp.float32)


_main_call = pl.pallas_call(
    _main_body,
    grid=(_B,),
    in_specs=[
        pl.BlockSpec((1, _N, _CODE_DIM), lambda b: (b, 0, 0)),
        pl.BlockSpec((1, _N, _CODE_DIM), lambda b: (b, 0, 0)),
        pl.BlockSpec((_HIDDEN, _CODE_DIM), lambda b: (0, 0)),
        pl.BlockSpec((1, _HIDDEN), lambda b: (0, 0)),
    ],
    out_specs=[
        pl.BlockSpec((1, _N, _HIDDEN), lambda b: (b, 0, 0)),
        pl.BlockSpec((1, _SUB, 128), lambda b: (b, 0, 0)),
        pl.BlockSpec((1, 1), lambda b: (0, 0)),
    ],
    out_shape=[
        jax.ShapeDtypeStruct((_B, _N, _HIDDEN), jnp.float32),
        jax.ShapeDtypeStruct((_B, _SUB, 128), jnp.int32),
        jax.ShapeDtypeStruct((1, 1), jnp.float32),
    ],
    scratch_shapes=[pltpu.SMEM((2,), jnp.float32)],
)


def kernel(mlc_proj, code, code_id, W, b):
    idx = code_id.reshape(_NW, _N_CH, _CH).astype(jnp.int32)
    quant_flat = _sc_gather(code, idx)
    quant = quant_flat.reshape(_B, _N, _CODE_DIM)
    out, valid3, loss = _main_call(quant, mlc_proj, W, b.reshape(1, _HIDDEN))
    valid = valid3.reshape(_B, _N) != 0
    return out, valid, loss.reshape(())
